# baseline (speedup 1.0000x reference)
import jax
import jax.numpy as jnp
from jax.experimental import pallas as pl
from jax.experimental.pallas import tpu as pltpu

_NB = 10
_C = 10
_N = 2_000_000
_W = 1280
_LW = _W // _C
_R = 1024
_CH = 8
_UNROLL = 4
_ITERS = _R // (_CH * _UNROLL)
_CUTS = (0, 499_968, 999_936, 1_499_904, _N)


def _make_partial(rows):
    nblk = -(-rows // _R)

    def _part_kernel(bins_ref, p_ref, l_ref, cnt, sm, ac, mt):
        pid = pl.program_id(0)
        k = pl.program_id(1)

        @pl.when(jnp.logical_and(pid == 0, k == 0))
        def _init():
            cnt[...] = jnp.zeros_like(cnt)
            sm[...] = jnp.zeros_like(sm)
            ac[...] = jnp.zeros_like(ac)

        @pl.when(k == 0)
        def _prep():
            lab = l_ref[...].astype(jnp.float32)
            u_iota = jax.lax.broadcasted_iota(jnp.int32, (_LW, _W), 0)
            j_grp = jax.lax.broadcasted_iota(jnp.int32, (_LW, _W), 1) // _C
            exp_mat = (u_iota == j_grp).astype(jnp.float32)
            l_w = jnp.dot(lab, exp_mat, preferred_element_type=jnp.float32)
            cls = (jax.lax.broadcasted_iota(jnp.int32, (_R, _W), 1)
                   % _C).astype(jnp.float32)
            mt[...] = (l_w == cls).astype(jnp.float32)

        e = bins_ref[0, k]
        limit = rows - pid * _R
        zed = jnp.zeros((_CH, _W), jnp.float32)
        row_iota = jax.lax.broadcasted_iota(jnp.int32, (_CH, _W), 0)

        def mk_body(masked):
            def body(i, carry):
                na, sa, aa = carry
                for t in range(_UNROLL):
                    base = (i * _UNROLL + t) * _CH
                    pc = p_ref[pl.ds(base, _CH), :]
                    mc = mt[pl.ds(base, _CH), :]
                    if masked:
                        pc = jnp.where(row_iota < limit - base, pc, 0.0)
                    gt = pc > e
                    na = na + jnp.where(gt, 1.0, 0.0)
                    sa = sa + jnp.where(gt, pc, 0.0)
                    aa = aa + jnp.where(gt, mc, 0.0)
                return na, sa, aa
            return body

        def run(masked):
            na, sa, aa = jax.lax.fori_loop(
                0, _ITERS, mk_body(masked), (zed, zed, zed))
            cnt[pl.ds(k, 1), :] += jnp.sum(na, axis=0, keepdims=True)
            sm[pl.ds(k, 1), :] += jnp.sum(sa, axis=0, keepdims=True)
            ac[pl.ds(k, 1), :] += jnp.sum(aa, axis=0, keepdims=True)

        is_last = pid == nblk - 1

        @pl.when(jnp.logical_not(is_last))
        def _fast():
            run(False)

        @pl.when(is_last)
        def _masked():
            run(True)

    def call(bins, pw, lw):
        shp = jax.ShapeDtypeStruct((_NB + 1, _W), jnp.float32)
        return pl.pallas_call(
            _part_kernel,
            grid=(nblk, _NB + 1),
            in_specs=[
                pl.BlockSpec(memory_space=pltpu.SMEM),
                pl.BlockSpec((_R, _W), lambda i, k: (i, 0)),
                pl.BlockSpec((_R, _LW), lambda i, k: (i, 0)),
            ],
            out_specs=[
                pl.BlockSpec((_NB + 1, _W), lambda i, k: (0, 0)),
                pl.BlockSpec((_NB + 1, _W), lambda i, k: (0, 0)),
                pl.BlockSpec((_NB + 1, _W), lambda i, k: (0, 0)),
            ],
            out_shape=[shp, shp, shp],
            scratch_shapes=[pltpu.VMEM((_R, _W), jnp.float32)],
            compiler_params=pltpu.CompilerParams(
                dimension_semantics=("arbitrary", "arbitrary"),
            ),
        )(bins, pw, lw)

    return call


def _fin_kernel(*refs):
    out_ref = refs[-1]
    parts = refs[:-1]
    cn = parts[0][...]
    sA = parts[1][...]
    aA = parts[2][...]
    for i in range(3, len(parts), 3):
        cn = cn + parts[i][...]
        sA = sA + parts[i + 1][...]
        aA = aA + parts[i + 2][...]
    n10 = cn[: _NB, :] - cn[1:, :]
    s10 = sA[: _NB, :] - sA[1:, :]
    a10 = aA[: _NB, :] - aA[1:, :]
    fc_j = jax.lax.broadcasted_iota(jnp.int32, (_W, 16), 0) % _C
    fc_c = jax.lax.broadcasted_iota(jnp.int32, (_W, 16), 1)
    foldc = (fc_j == fc_c).astype(jnp.float32)
    nf = jnp.dot(n10, foldc, preferred_element_type=jnp.float32)
    sf = jnp.dot(s10, foldc, preferred_element_type=jnp.float32)
    af = jnp.dot(a10, foldc, preferred_element_type=jnp.float32)
    nonempty = nf > 0
    safe_n = jnp.where(nonempty, nf, 1.0)
    d = sf - af
    term = jnp.where(nonempty, d * d / safe_n, 0.0)
    tot = jnp.sum(nf, axis=0, keepdims=True)
    cep = jnp.sum(term, axis=0, keepdims=True) / jnp.where(
        tot > 0, tot, 1.0)
    tot_cep = jnp.sum(cep, axis=1, keepdims=True)
    out_ref[...] = jnp.sqrt(tot_cep / _C)


def kernel(probas, labels):
    bins = jnp.linspace(0.0, 1.0, _NB + 1).reshape(1, _NB + 1)
    partials = []
    for lo, hi in zip(_CUTS, _CUTS[1:]):
        rows = (hi - lo) * _C // _W
        pw = probas[lo:hi].reshape(rows, _W)
        lw = labels[lo:hi].reshape(rows, _LW)
        partials.extend(_make_partial(rows)(bins, pw, lw))
    out = pl.pallas_call(
        _fin_kernel,
        out_shape=jax.ShapeDtypeStruct((1, 1), jnp.float32),
    )(*partials)
    return out.reshape(())


# speedup vs baseline: 1.2845x; 1.2845x over previous
import jax
import jax.numpy as jnp
from jax.experimental import pallas as pl
from jax.experimental.pallas import tpu as pltpu

_NB = 10
_C = 10
_N = 2_000_000
_W = 1280
_LW = _W // _C
_R = 1024
_CH = 8
_UNROLL = 4
_ITERS = _R // (_CH * _UNROLL)
_CUTS = (0, 999_936, _N)


def _make_partial(rows):
    nblk = -(-rows // _R)

    def _part_kernel(bins_ref, p_ref, l_ref, cnt, sm, ac, mt):
        pid = pl.program_id(0)
        k = pl.program_id(1)

        @pl.when(jnp.logical_and(pid == 0, k == 0))
        def _init():
            cnt[...] = jnp.zeros_like(cnt)
            sm[...] = jnp.zeros_like(sm)
            ac[...] = jnp.zeros_like(ac)

        @pl.when(k == 0)
        def _prep():
            lab = l_ref[...].astype(jnp.float32)
            u_iota = jax.lax.broadcasted_iota(jnp.int32, (_LW, _W), 0)
            j_grp = jax.lax.broadcasted_iota(jnp.int32, (_LW, _W), 1) // _C
            exp_mat = (u_iota == j_grp).astype(jnp.float32)
            l_w = jnp.dot(lab, exp_mat, preferred_element_type=jnp.float32)
            cls = (jax.lax.broadcasted_iota(jnp.int32, (_R, _W), 1)
                   % _C).astype(jnp.float32)
            mt[...] = (l_w == cls).astype(jnp.float32)

        e = bins_ref[0, k]
        limit = rows - pid * _R
        zed = jnp.zeros((_CH, _W), jnp.float32)
        row_iota = jax.lax.broadcasted_iota(jnp.int32, (_CH, _W), 0)

        def mk_body(masked):
            def body(i, carry):
                na, sa, aa = carry
                for t in range(_UNROLL):
                    base = (i * _UNROLL + t) * _CH
                    pc = p_ref[pl.ds(base, _CH), :]
                    mc = mt[pl.ds(base, _CH), :]
                    if masked:
                        pc = jnp.where(row_iota < limit - base, pc, 0.0)
                    gt = pc > e
                    na = na + jnp.where(gt, 1.0, 0.0)
                    sa = sa + jnp.where(gt, pc, 0.0)
                    aa = aa + jnp.where(gt, mc, 0.0)
                return na, sa, aa
            return body

        def run(masked):
            na, sa, aa = jax.lax.fori_loop(
                0, _ITERS, mk_body(masked), (zed, zed, zed))
            cnt[pl.ds(k, 1), :] += jnp.sum(na, axis=0, keepdims=True)
            sm[pl.ds(k, 1), :] += jnp.sum(sa, axis=0, keepdims=True)
            ac[pl.ds(k, 1), :] += jnp.sum(aa, axis=0, keepdims=True)

        is_last = pid == nblk - 1

        @pl.when(jnp.logical_not(is_last))
        def _fast():
            run(False)

        @pl.when(is_last)
        def _masked():
            run(True)

    def call(bins, pw, lw):
        shp = jax.ShapeDtypeStruct((_NB + 1, _W), jnp.float32)
        return pl.pallas_call(
            _part_kernel,
            grid=(nblk, _NB + 1),
            in_specs=[
                pl.BlockSpec(memory_space=pltpu.SMEM),
                pl.BlockSpec((_R, _W), lambda i, k: (i, 0)),
                pl.BlockSpec((_R, _LW), lambda i, k: (i, 0)),
            ],
            out_specs=[
                pl.BlockSpec((_NB + 1, _W), lambda i, k: (0, 0)),
                pl.BlockSpec((_NB + 1, _W), lambda i, k: (0, 0)),
                pl.BlockSpec((_NB + 1, _W), lambda i, k: (0, 0)),
            ],
            out_shape=[shp, shp, shp],
            scratch_shapes=[pltpu.VMEM((_R, _W), jnp.float32)],
            compiler_params=pltpu.CompilerParams(
                dimension_semantics=("arbitrary", "arbitrary"),
            ),
        )(bins, pw, lw)

    return call


def _fin_kernel(*refs):
    out_ref = refs[-1]
    parts = refs[:-1]
    cn = parts[0][...]
    sA = parts[1][...]
    aA = parts[2][...]
    for i in range(3, len(parts), 3):
        cn = cn + parts[i][...]
        sA = sA + parts[i + 1][...]
        aA = aA + parts[i + 2][...]
    n10 = cn[: _NB, :] - cn[1:, :]
    s10 = sA[: _NB, :] - sA[1:, :]
    a10 = aA[: _NB, :] - aA[1:, :]
    fc_j = jax.lax.broadcasted_iota(jnp.int32, (_W, 16), 0) % _C
    fc_c = jax.lax.broadcasted_iota(jnp.int32, (_W, 16), 1)
    foldc = (fc_j == fc_c).astype(jnp.float32)
    nf = jnp.dot(n10, foldc, preferred_element_type=jnp.float32)
    sf = jnp.dot(s10, foldc, preferred_element_type=jnp.float32)
    af = jnp.dot(a10, foldc, preferred_element_type=jnp.float32)
    nonempty = nf > 0
    safe_n = jnp.where(nonempty, nf, 1.0)
    d = sf - af
    term = jnp.where(nonempty, d * d / safe_n, 0.0)
    tot = jnp.sum(nf, axis=0, keepdims=True)
    cep = jnp.sum(term, axis=0, keepdims=True) / jnp.where(
        tot > 0, tot, 1.0)
    tot_cep = jnp.sum(cep, axis=1, keepdims=True)
    out_ref[...] = jnp.sqrt(tot_cep / _C)


def kernel(probas, labels):
    bins = jnp.linspace(0.0, 1.0, _NB + 1).reshape(1, _NB + 1)
    partials = []
    for lo, hi in zip(_CUTS, _CUTS[1:]):
        rows = (hi - lo) * _C // _W
        pw = probas[lo:hi].reshape(rows, _W)
        lw = labels[lo:hi].reshape(rows, _LW)
        partials.extend(_make_partial(rows)(bins, pw, lw))
    out = pl.pallas_call(
        _fin_kernel,
        out_shape=jax.ShapeDtypeStruct((1, 1), jnp.float32),
    )(*partials)
    return out.reshape(())


# speedup vs baseline: 1.3039x; 1.0151x over previous
import jax
import jax.numpy as jnp
from jax.experimental import pallas as pl
from jax.experimental.pallas import tpu as pltpu

_NB = 10
_C = 10
_N = 2_000_000
_W = 1280
_LW = _W // _C
_R = 1024
_CH = 8
_UNROLL = 8
_ITERS = _R // (_CH * _UNROLL)
_CUTS = (0, 999_936, _N)


def _make_partial(rows):
    nblk = -(-rows // _R)

    def _part_kernel(bins_ref, p_ref, l_ref, cnt, sm, ac, mt):
        pid = pl.program_id(0)
        k = pl.program_id(1)

        @pl.when(jnp.logical_and(pid == 0, k == 0))
        def _init():
            cnt[...] = jnp.zeros_like(cnt)
            sm[...] = jnp.zeros_like(sm)
            ac[...] = jnp.zeros_like(ac)

        @pl.when(k == 0)
        def _prep():
            lab = l_ref[...].astype(jnp.float32)
            u_iota = jax.lax.broadcasted_iota(jnp.int32, (_LW, _W), 0)
            j_grp = jax.lax.broadcasted_iota(jnp.int32, (_LW, _W), 1) // _C
            exp_mat = (u_iota == j_grp).astype(jnp.float32)
            l_w = jnp.dot(lab, exp_mat, preferred_element_type=jnp.float32)
            cls = (jax.lax.broadcasted_iota(jnp.int32, (_R, _W), 1)
                   % _C).astype(jnp.float32)
            mt[...] = (l_w == cls).astype(jnp.float32)

        e = bins_ref[0, k]
        limit = rows - pid * _R
        zed = jnp.zeros((_CH, _W), jnp.float32)
        row_iota = jax.lax.broadcasted_iota(jnp.int32, (_CH, _W), 0)

        def mk_body(masked):
            def body(i, carry):
                na, sa, aa = carry
                for t in range(_UNROLL):
                    base = (i * _UNROLL + t) * _CH
                    pc = p_ref[pl.ds(base, _CH), :]
                    mc = mt[pl.ds(base, _CH), :]
                    if masked:
                        pc = jnp.where(row_iota < limit - base, pc, 0.0)
                    gt = pc > e
                    na = na + jnp.where(gt, 1.0, 0.0)
                    sa = sa + jnp.where(gt, pc, 0.0)
                    aa = aa + jnp.where(gt, mc, 0.0)
                return na, sa, aa
            return body

        def run(masked):
            na, sa, aa = jax.lax.fori_loop(
                0, _ITERS, mk_body(masked), (zed, zed, zed))
            cnt[pl.ds(k, 1), :] += jnp.sum(na, axis=0, keepdims=True)
            sm[pl.ds(k, 1), :] += jnp.sum(sa, axis=0, keepdims=True)
            ac[pl.ds(k, 1), :] += jnp.sum(aa, axis=0, keepdims=True)

        is_last = pid == nblk - 1

        @pl.when(jnp.logical_not(is_last))
        def _fast():
            run(False)

        @pl.when(is_last)
        def _masked():
            run(True)

    def call(bins, pw, lw):
        shp = jax.ShapeDtypeStruct((_NB + 1, _W), jnp.float32)
        return pl.pallas_call(
            _part_kernel,
            grid=(nblk, _NB + 1),
            in_specs=[
                pl.BlockSpec(memory_space=pltpu.SMEM),
                pl.BlockSpec((_R, _W), lambda i, k: (i, 0)),
                pl.BlockSpec((_R, _LW), lambda i, k: (i, 0)),
            ],
            out_specs=[
                pl.BlockSpec((_NB + 1, _W), lambda i, k: (0, 0)),
                pl.BlockSpec((_NB + 1, _W), lambda i, k: (0, 0)),
                pl.BlockSpec((_NB + 1, _W), lambda i, k: (0, 0)),
            ],
            out_shape=[shp, shp, shp],
            scratch_shapes=[pltpu.VMEM((_R, _W), jnp.float32)],
            compiler_params=pltpu.CompilerParams(
                dimension_semantics=("arbitrary", "arbitrary"),
            ),
        )(bins, pw, lw)

    return call


def _fin_kernel(*refs):
    out_ref = refs[-1]
    parts = refs[:-1]
    cn = parts[0][...]
    sA = parts[1][...]
    aA = parts[2][...]
    for i in range(3, len(parts), 3):
        cn = cn + parts[i][...]
        sA = sA + parts[i + 1][...]
        aA = aA + parts[i + 2][...]
    n10 = cn[: _NB, :] - cn[1:, :]
    s10 = sA[: _NB, :] - sA[1:, :]
    a10 = aA[: _NB, :] - aA[1:, :]
    fc_j = jax.lax.broadcasted_iota(jnp.int32, (_W, 16), 0) % _C
    fc_c = jax.lax.broadcasted_iota(jnp.int32, (_W, 16), 1)
    foldc = (fc_j == fc_c).astype(jnp.float32)
    nf = jnp.dot(n10, foldc, preferred_element_type=jnp.float32)
    sf = jnp.dot(s10, foldc, preferred_element_type=jnp.float32)
    af = jnp.dot(a10, foldc, preferred_element_type=jnp.float32)
    nonempty = nf > 0
    safe_n = jnp.where(nonempty, nf, 1.0)
    d = sf - af
    term = jnp.where(nonempty, d * d / safe_n, 0.0)
    tot = jnp.sum(nf, axis=0, keepdims=True)
    cep = jnp.sum(term, axis=0, keepdims=True) / jnp.where(
        tot > 0, tot, 1.0)
    tot_cep = jnp.sum(cep, axis=1, keepdims=True)
    out_ref[...] = jnp.sqrt(tot_cep / _C)


def kernel(probas, labels):
    bins = jnp.linspace(0.0, 1.0, _NB + 1).reshape(1, _NB + 1)
    partials = []
    for lo, hi in zip(_CUTS, _CUTS[1:]):
        rows = (hi - lo) * _C // _W
        pw = probas[lo:hi].reshape(rows, _W)
        lw = labels[lo:hi].reshape(rows, _LW)
        partials.extend(_make_partial(rows)(bins, pw, lw))
    out = pl.pallas_call(
        _fin_kernel,
        out_shape=jax.ShapeDtypeStruct((1, 1), jnp.float32),
    )(*partials)
    return out.reshape(())
